# Initial kernel scaffold; baseline (speedup 1.0000x reference)
import jax
import jax.numpy as jnp
from jax.experimental import pallas as pl


def kernel(x):
    raise NotImplementedError("write your pallas kernel here")


# speedup vs baseline: 1.8838x; 1.8838x over previous
import functools

import jax
import jax.numpy as jnp
from jax import lax
from jax.experimental import pallas as pl
from jax.experimental.pallas import tpu as pltpu
from jax.experimental.pallas import tpu_sc as plsc

_B = 16384
_C = 512
_N = _B * _C
_EPS = 1e-8

_NC = 2
_NS = 16
_NW = _NC * _NS
_PER_W = _N // _NW
_CHUNK = 32768
_NCHUNK = _PER_W // _CHUNK
_GROUPS_PER_CHUNK = _CHUNK // 32


@functools.partial(
    pl.kernel,
    out_type=jax.ShapeDtypeStruct((_N,), jnp.float32),
    mesh=plsc.VectorSubcoreMesh(core_axis_name="c", subcore_axis_name="s"),
    scratch_types=[
        pltpu.VMEM((_CHUNK,), jnp.float32),
    ],
    compiler_params=pltpu.CompilerParams(needs_layout_passes=False),
)
def _sc_group_softmax(x_hbm, out_hbm, buf):
    wid = lax.axis_index("s") * _NC + lax.axis_index("c")
    base = wid * _PER_W

    def chunk_body(c, carry):
        off = pl.multiple_of(base + c * _CHUNK, _CHUNK)
        pltpu.sync_copy(x_hbm.at[pl.ds(off, _CHUNK)], buf)

        def group_body(g, carry2):
            o = pl.multiple_of(g * 32, 32)
            a = buf[pl.ds(o, 16)]
            b = buf[pl.ds(o + 16, 16)]
            m = jnp.max(jnp.maximum(a, b))
            ea = jnp.exp(a - m)
            eb = jnp.exp(b - m)
            dvec = lax.broadcast(jnp.sum(ea + eb) + _EPS, (16,))
            r = jnp.full((16,), 1.0, jnp.float32) / dvec
            buf[pl.ds(o, 16)] = ea * r
            buf[pl.ds(o + 16, 16)] = eb * r
            return carry2

        lax.fori_loop(0, _GROUPS_PER_CHUNK, group_body, 0, unroll=4)
        pltpu.sync_copy(buf, out_hbm.at[pl.ds(off, _CHUNK)])
        return carry

    lax.fori_loop(0, _NCHUNK, chunk_body, 0)


def kernel(x):
    xf = x.reshape(_N)
    out = _sc_group_softmax(xf)
    return out.reshape(_B, _C, 1)


# speedup vs baseline: 2.2158x; 1.1763x over previous
import functools

import jax
import jax.numpy as jnp
from jax import lax
from jax.experimental import pallas as pl
from jax.experimental.pallas import tpu as pltpu
from jax.experimental.pallas import tpu_sc as plsc

_B = 16384
_C = 512
_N = _B * _C
_EPS = 1e-8

_NC = 2
_NS = 16
_NW = _NC * _NS
_PER_W = _N // _NW
_CHUNK = 32768
_NCHUNK = _PER_W // _CHUNK
_GROUPS_PER_CHUNK = _CHUNK // 32
_NBUF = 3


@functools.partial(
    pl.kernel,
    out_type=jax.ShapeDtypeStruct((_N,), jnp.float32),
    mesh=plsc.VectorSubcoreMesh(core_axis_name="c", subcore_axis_name="s"),
    scratch_types=(
        [pltpu.VMEM((_CHUNK,), jnp.float32) for _ in range(_NBUF)]
        + [pltpu.SemaphoreType.DMA for _ in range(2 * _NBUF)]
    ),
    compiler_params=pltpu.CompilerParams(needs_layout_passes=False),
)
def _sc_group_softmax(x_hbm, out_hbm, b0, b1, b2, si0, si1, si2, so0, so1, so2):
    bufs = (b0, b1, b2)
    sin = (si0, si1, si2)
    sout = (so0, so1, so2)
    wid = lax.axis_index("s") * _NC + lax.axis_index("c")
    base = wid * _PER_W

    def in_copy(ci):
        p = ci % _NBUF
        off = pl.multiple_of(base + ci * _CHUNK, _CHUNK)
        return pltpu.make_async_copy(x_hbm.at[pl.ds(off, _CHUNK)], bufs[p], sin[p])

    def out_copy(ci):
        p = ci % _NBUF
        off = pl.multiple_of(base + ci * _CHUNK, _CHUNK)
        return pltpu.make_async_copy(bufs[p], out_hbm.at[pl.ds(off, _CHUNK)], sout[p])

    def compute(buf):
        def group_body(g, carry):
            o = pl.multiple_of(g * 32, 32)
            a = buf[pl.ds(o, 16)]
            b = buf[pl.ds(o + 16, 16)]
            m = jnp.max(jnp.maximum(a, b))
            ea = jnp.exp(a - m)
            eb = jnp.exp(b - m)
            dvec = lax.broadcast(jnp.sum(ea + eb) + _EPS, (16,))
            r = jnp.full((16,), 1.0, jnp.float32) / dvec
            buf[pl.ds(o, 16)] = ea * r
            buf[pl.ds(o + 16, 16)] = eb * r
            return carry

        lax.fori_loop(0, _GROUPS_PER_CHUNK, group_body, 0, unroll=4)

    in_copy(0).start()
    for ci in range(_NCHUNK):
        if ci + 1 < _NCHUNK:
            if ci >= 2:
                out_copy(ci - 2).wait()
            in_copy(ci + 1).start()
        in_copy(ci).wait()
        compute(bufs[ci % _NBUF])
        out_copy(ci).start()
    out_copy(_NCHUNK - 2).wait()
    out_copy(_NCHUNK - 1).wait()


def kernel(x):
    xf = x.reshape(_N)
    out = _sc_group_softmax(xf)
    return out.reshape(_B, _C, 1)


# speedup vs baseline: 3.4851x; 1.5728x over previous
import functools

import jax
import jax.numpy as jnp
from jax import lax
from jax.experimental import pallas as pl
from jax.experimental.pallas import tpu as pltpu
from jax.experimental.pallas import tpu_sc as plsc

_B = 16384
_C = 512
_N = _B * _C
_EPS = 1e-8

_NC = 2
_NS = 16
_NW = _NC * _NS
_PER_W = _N // _NW
_CHUNK = 32768
_NCHUNK = _PER_W // _CHUNK
_GROUPS_PER_CHUNK = _CHUNK // 32
_NBUF = 3


@functools.partial(
    pl.kernel,
    out_type=jax.ShapeDtypeStruct((_N,), jnp.float32),
    mesh=plsc.VectorSubcoreMesh(core_axis_name="c", subcore_axis_name="s"),
    scratch_types=(
        [pltpu.VMEM((_CHUNK,), jnp.float32) for _ in range(_NBUF)]
        + [pltpu.SemaphoreType.DMA for _ in range(2 * _NBUF)]
    ),
    compiler_params=pltpu.CompilerParams(needs_layout_passes=False),
)
def _sc_group_softmax(x_hbm, out_hbm, b0, b1, b2, si0, si1, si2, so0, so1, so2):
    bufs = (b0, b1, b2)
    sin = (si0, si1, si2)
    sout = (so0, so1, so2)
    wid = lax.axis_index("s") * _NC + lax.axis_index("c")
    base = wid * _PER_W

    def in_copy(ci):
        p = ci % _NBUF
        off = pl.multiple_of(base + ci * _CHUNK, _CHUNK)
        return pltpu.make_async_copy(x_hbm.at[pl.ds(off, _CHUNK)], bufs[p], sin[p])

    def out_copy(ci):
        p = ci % _NBUF
        off = pl.multiple_of(base + ci * _CHUNK, _CHUNK)
        return pltpu.make_async_copy(bufs[p], out_hbm.at[pl.ds(off, _CHUNK)], sout[p])

    def compute(buf):
        def group_body(g, carry):
            o = pl.multiple_of(g * 32, 32)
            a = buf[pl.ds(o, 16)]
            b = buf[pl.ds(o + 16, 16)]
            m = jnp.max(jnp.maximum(a, b))
            ea = jnp.exp(a - m)
            eb = jnp.exp(b - m)
            dvec = lax.broadcast(jnp.sum(ea + eb) + _EPS, (16,))
            r = jnp.full((16,), 1.0, jnp.float32) / dvec
            buf[pl.ds(o, 16)] = ea * r
            buf[pl.ds(o + 16, 16)] = eb * r
            return carry

        lax.fori_loop(0, _GROUPS_PER_CHUNK, group_body, 0, unroll=8)

    in_copy(0).start()
    for ci in range(_NCHUNK):
        if ci + 1 < _NCHUNK:
            if ci >= 2:
                out_copy(ci - 2).wait()
            in_copy(ci + 1).start()
        in_copy(ci).wait()
        compute(bufs[ci % _NBUF])
        out_copy(ci).start()
    out_copy(_NCHUNK - 2).wait()
    out_copy(_NCHUNK - 1).wait()


def kernel(x):
    xf = x.reshape(_N)
    out = _sc_group_softmax(xf)
    return out.reshape(_B, _C, 1)


# speedup vs baseline: 3.4916x; 1.0019x over previous
import functools

import jax
import jax.numpy as jnp
from jax import lax
from jax.experimental import pallas as pl
from jax.experimental.pallas import tpu as pltpu
from jax.experimental.pallas import tpu_sc as plsc

_B = 16384
_C = 512
_N = _B * _C
_EPS = 1e-8

_NC = 2
_NS = 16
_NW = _NC * _NS
_PER_W = _N // _NW
_CHUNK = 32768
_NCHUNK = _PER_W // _CHUNK
_GROUPS_PER_CHUNK = _CHUNK // 32
_NBUF = 3


@functools.partial(
    pl.kernel,
    out_type=jax.ShapeDtypeStruct((_N,), jnp.float32),
    mesh=plsc.VectorSubcoreMesh(core_axis_name="c", subcore_axis_name="s"),
    scratch_types=(
        [pltpu.VMEM((_CHUNK,), jnp.float32) for _ in range(_NBUF)]
        + [pltpu.SemaphoreType.DMA for _ in range(2 * _NBUF)]
    ),
    compiler_params=pltpu.CompilerParams(needs_layout_passes=False),
)
def _sc_group_softmax(x_hbm, out_hbm, b0, b1, b2, si0, si1, si2, so0, so1, so2):
    bufs = (b0, b1, b2)
    sin = (si0, si1, si2)
    sout = (so0, so1, so2)
    wid = lax.axis_index("s") * _NC + lax.axis_index("c")
    base = wid * _PER_W

    def in_copy(ci):
        p = ci % _NBUF
        off = pl.multiple_of(base + ci * _CHUNK, _CHUNK)
        return pltpu.make_async_copy(x_hbm.at[pl.ds(off, _CHUNK)], bufs[p], sin[p])

    def out_copy(ci):
        p = ci % _NBUF
        off = pl.multiple_of(base + ci * _CHUNK, _CHUNK)
        return pltpu.make_async_copy(bufs[p], out_hbm.at[pl.ds(off, _CHUNK)], sout[p])

    def compute(buf):
        def group_body(g, carry):
            o = pl.multiple_of(g * 32, 32)
            a = buf[pl.ds(o, 16)]
            b = buf[pl.ds(o + 16, 16)]
            m = jnp.max(jnp.maximum(a, b))
            ea = jnp.exp(a - m)
            eb = jnp.exp(b - m)
            dvec = lax.broadcast(jnp.sum(ea + eb) + _EPS, (16,))
            r = jnp.full((16,), 1.0, jnp.float32) / dvec
            buf[pl.ds(o, 16)] = ea * r
            buf[pl.ds(o + 16, 16)] = eb * r
            return carry

        lax.fori_loop(0, _GROUPS_PER_CHUNK, group_body, 0, unroll=16)

    in_copy(0).start()
    for ci in range(_NCHUNK):
        if ci + 1 < _NCHUNK:
            if ci >= 2:
                out_copy(ci - 2).wait()
            in_copy(ci + 1).start()
        in_copy(ci).wait()
        compute(bufs[ci % _NBUF])
        out_copy(ci).start()
    out_copy(_NCHUNK - 2).wait()
    out_copy(_NCHUNK - 1).wait()


def kernel(x):
    xf = x.reshape(_N)
    out = _sc_group_softmax(xf)
    return out.reshape(_B, _C, 1)
